# baseline (device time: 340302 ns/iter reference)
import jax
import jax.numpy as jnp
from jax import lax
from jax.experimental import pallas as pl
from jax.experimental.pallas import tpu as pltpu

N_DEV = 8
P = 10
HOPS = N_DEV - 1


def kernel(x, Win0, Wout0, Win1, Wout1, Win2, Wout2):
    b, d = x.shape
    _, f_sh = Win0.shape
    assert Wout0.shape == (f_sh, d)

    def body(x_ref, win0_ref, wout0_ref, win1_ref, wout1_ref, win2_ref,
             wout2_ref, out_ref, cwin_ref, cwout_ref, acc_a, acc_b,
             swin, rwin, swout, rwout):
        me = lax.axis_index("i")
        right = lax.rem(me + 1, N_DEV)
        left = lax.rem(me - 1 + N_DEV, N_DEV)

        bar = pltpu.get_barrier_semaphore()
        for nbr in (left, right):
            pl.semaphore_signal(
                bar, inc=1, device_id=(nbr,),
                device_id_type=pl.DeviceIdType.MESH,
            )
        pl.semaphore_wait(bar, 2)

        def hop_send(g, win_src, wout_src):
            s = g % P
            rw = pltpu.make_async_remote_copy(
                src_ref=win_src, dst_ref=cwin_ref.at[s],
                send_sem=swin.at[s], recv_sem=rwin.at[s],
                device_id=(right,), device_id_type=pl.DeviceIdType.MESH,
            )
            ro = pltpu.make_async_remote_copy(
                src_ref=wout_src, dst_ref=cwout_ref.at[s],
                send_sem=swout.at[s], recv_sem=rwout.at[s],
                device_id=(right,), device_id_type=pl.DeviceIdType.MESH,
            )
            rw.start()
            ro.start()
            return rw, ro

        layers = (
            (win0_ref, wout0_ref, x_ref, acc_a),
            (win1_ref, wout1_ref, acc_a, acc_b),
            (win2_ref, wout2_ref, acc_b, acc_a),
        )
        for L, (win_ref, wout_ref, xs, acc) in enumerate(layers):
            for h in range(N_DEV):
                g = HOPS * L + h
                if h == 0:
                    win_blk, wout_blk = win_ref, wout_ref
                else:
                    win_blk = cwin_ref.at[(g - 1) % P]
                    wout_blk = cwout_ref.at[(g - 1) % P]
                rdmas = hop_send(g, win_blk, wout_blk) if h < HOPS else None
                hid = jnp.maximum(
                    jnp.dot(xs[...], win_blk[...],
                            preferred_element_type=jnp.float32),
                    0.0,
                )
                term = jnp.dot(hid, wout_blk[...],
                               preferred_element_type=jnp.float32)
                if h == 0:
                    acc[...] = term
                else:
                    acc[...] = acc[...] + term
                if rdmas is not None:
                    rdmas[0].wait()
                    rdmas[1].wait()

        xfin = acc_a
        out_ref[pl.ds(me * b, b), :] = xfin[...]
        for h in range(HOPS):
            g = 3 * HOPS + h
            src = xfin if h == 0 else cwout_ref.at[(g - 1) % P]
            r = pltpu.make_async_remote_copy(
                src_ref=src, dst_ref=cwout_ref.at[g % P],
                send_sem=swout.at[g % P], recv_sem=rwout.at[g % P],
                device_id=(right,), device_id_type=pl.DeviceIdType.MESH,
            )
            r.start()
            r.wait()
            origin = lax.rem(me - (h + 1) + N_DEV, N_DEV)
            out_ref[pl.ds(origin * b, b), :] = cwout_ref[g % P]

    return pl.pallas_call(
        body,
        out_shape=jax.ShapeDtypeStruct((N_DEV * b, d), jnp.float32),
        in_specs=[pl.BlockSpec(memory_space=pltpu.VMEM)] * 7,
        out_specs=pl.BlockSpec(memory_space=pltpu.VMEM),
        scratch_shapes=[
            pltpu.VMEM((P, d, f_sh), jnp.float32),
            pltpu.VMEM((P, f_sh, d), jnp.float32),
            pltpu.VMEM((b, d), jnp.float32),
            pltpu.VMEM((b, d), jnp.float32),
            pltpu.SemaphoreType.DMA((P,)),
            pltpu.SemaphoreType.DMA((P,)),
            pltpu.SemaphoreType.DMA((P,)),
            pltpu.SemaphoreType.DMA((P,)),
        ],
        compiler_params=pltpu.CompilerParams(collective_id=0),
    )(x, Win0, Wout0, Win1, Wout1, Win2, Wout2)


# device time: 115421 ns/iter; 2.9484x vs baseline; 2.9484x over previous
import jax
import jax.numpy as jnp
from jax import lax
from jax.experimental import pallas as pl
from jax.experimental.pallas import tpu as pltpu

N_DEV = 8
NL = 3


def kernel(x, Win0, Wout0, Win1, Wout1, Win2, Wout2):
    b, d = x.shape
    _, f = Win0.shape
    assert Wout0.shape == (f, d)
    bf16 = jnp.bfloat16

    def body(x_ref, win0_ref, wout0_ref, win1_ref, wout1_ref, win2_ref,
             wout2_ref, out_ref,
             mywin, mywout, wbuf, wobuf, agbuf, ag_mine, acc, xb,
             sw, rw, swo, rwo, sag, rag):
        me = lax.axis_index("i")

        win_refs = (win0_ref, win1_ref, win2_ref)
        wout_refs = (wout0_ref, wout1_ref, wout2_ref)
        for L in range(NL):
            mywin[L, :, :] = win_refs[L][...].astype(bf16)
            mywout[L, :, :] = wout_refs[L][...].astype(bf16)
        xb[0, :, :] = x_ref[...].astype(bf16)

        bar = pltpu.get_barrier_semaphore()
        for dd in range(1, N_DEV):
            pl.semaphore_signal(
                bar, inc=1, device_id=(lax.rem(me + dd, N_DEV),),
                device_id_type=pl.DeviceIdType.MESH,
            )
        pl.semaphore_wait(bar, N_DEV - 1)

        wsends = []
        for L in range(NL):
            for dd in range(1, N_DEV):
                tgt = (lax.rem(me + dd, N_DEV),)
                dw = pltpu.make_async_remote_copy(
                    src_ref=mywin.at[L], dst_ref=wbuf.at[L, dd - 1],
                    send_sem=sw.at[L, dd - 1], recv_sem=rw.at[L, dd - 1],
                    device_id=tgt, device_id_type=pl.DeviceIdType.MESH,
                )
                do = pltpu.make_async_remote_copy(
                    src_ref=mywout.at[L], dst_ref=wobuf.at[L, dd - 1],
                    send_sem=swo.at[L, dd - 1], recv_sem=rwo.at[L, dd - 1],
                    device_id=tgt, device_id_type=pl.DeviceIdType.MESH,
                )
                dw.start()
                do.start()
                wsends.append((dw, do))

        for L in range(NL):
            xcur = xb[L, :, :]
            hid = jnp.maximum(
                jnp.dot(xcur, mywin[L, :, :],
                        preferred_element_type=jnp.float32), 0.0)
            acc[...] = jnp.dot(hid.astype(bf16), mywout[L, :, :],
                               preferred_element_type=jnp.float32)
            for dd in range(1, N_DEV):
                wsends[L * (N_DEV - 1) + dd - 1][0].wait_recv()
                wsends[L * (N_DEV - 1) + dd - 1][1].wait_recv()
                hid = jnp.maximum(
                    jnp.dot(xcur, wbuf[L, dd - 1, :, :],
                            preferred_element_type=jnp.float32), 0.0)
                acc[...] = acc[...] + jnp.dot(
                    hid.astype(bf16), wobuf[L, dd - 1, :, :],
                    preferred_element_type=jnp.float32)
            if L < NL - 1:
                xb[L + 1, :, :] = acc[...].astype(bf16)

        ag_mine[...] = acc[...].astype(bf16)
        out_ref[pl.ds(me * b, b), :] = acc[...]
        agsends = []
        for dd in range(1, N_DEV):
            r = pltpu.make_async_remote_copy(
                src_ref=ag_mine, dst_ref=agbuf.at[dd - 1],
                send_sem=sag.at[dd - 1], recv_sem=rag.at[dd - 1],
                device_id=(lax.rem(me + dd, N_DEV),),
                device_id_type=pl.DeviceIdType.MESH,
            )
            r.start()
            agsends.append(r)
        for dd in range(1, N_DEV):
            agsends[dd - 1].wait_recv()
            origin = lax.rem(me - dd + N_DEV, N_DEV)
            out_ref[pl.ds(origin * b, b), :] = agbuf[dd - 1, :, :].astype(
                jnp.float32)

        for dw, do in wsends:
            dw.wait_send()
            do.wait_send()
        for r in agsends:
            r.wait_send()

    return pl.pallas_call(
        body,
        out_shape=jax.ShapeDtypeStruct((N_DEV * b, d), jnp.float32),
        in_specs=[pl.BlockSpec(memory_space=pltpu.VMEM)] * 7,
        out_specs=pl.BlockSpec(memory_space=pltpu.VMEM),
        scratch_shapes=[
            pltpu.VMEM((NL, d, f), bf16),
            pltpu.VMEM((NL, f, d), bf16),
            pltpu.VMEM((NL, N_DEV - 1, d, f), bf16),
            pltpu.VMEM((NL, N_DEV - 1, f, d), bf16),
            pltpu.VMEM((N_DEV - 1, b, d), bf16),
            pltpu.VMEM((b, d), bf16),
            pltpu.VMEM((b, d), jnp.float32),
            pltpu.VMEM((NL, b, d), bf16),
            pltpu.SemaphoreType.DMA((NL, N_DEV - 1)),
            pltpu.SemaphoreType.DMA((NL, N_DEV - 1)),
            pltpu.SemaphoreType.DMA((NL, N_DEV - 1)),
            pltpu.SemaphoreType.DMA((NL, N_DEV - 1)),
            pltpu.SemaphoreType.DMA((N_DEV - 1,)),
            pltpu.SemaphoreType.DMA((N_DEV - 1,)),
        ],
        compiler_params=pltpu.CompilerParams(collective_id=0),
    )(x, Win0, Wout0, Win1, Wout1, Win2, Wout2)


# device time: 81908 ns/iter; 4.1547x vs baseline; 1.4092x over previous
import jax
import jax.numpy as jnp
from jax import lax
from jax.experimental import pallas as pl
from jax.experimental.pallas import tpu as pltpu

N_DEV = 8
NL = 3
DIM_MASKS = (1, 3, 4)
LAYER_DIMS = tuple(
    tuple(DIM_MASKS[(L + p) % 3] for p in range(3)) for L in range(NL)
)


def kernel(x, Win0, Wout0, Win1, Wout1, Win2, Wout2):
    b, d = x.shape
    _, f = Win0.shape
    assert Wout0.shape == (f, d)
    bf16 = jnp.bfloat16

    def body(x_ref, win0_ref, wout0_ref, win1_ref, wout1_ref, win2_ref,
             wout2_ref, out_ref,
             wq, woq, agbuf, ag_mine, xb,
             sw, rw, swo, rwo, sag, rag):
        me = lax.axis_index("i")

        win_refs = (win0_ref, win1_ref, win2_ref)
        wout_refs = (wout0_ref, wout1_ref, wout2_ref)
        for L in range(NL):
            wq[L, 0, :, :] = win_refs[L][...].astype(bf16)
            woq[L, 0, :, :] = wout_refs[L][...].astype(bf16)
        xb[0, :, :] = x_ref[...].astype(bf16)

        bar = pltpu.get_barrier_semaphore()
        for dd in range(1, N_DEV):
            pl.semaphore_signal(
                bar, inc=1, device_id=(lax.rem(me + dd, N_DEV),),
                device_id_type=pl.DeviceIdType.MESH,
            )
        pl.semaphore_wait(bar, N_DEV - 1)

        def make_phase(L, p):
            n = 1 << p
            tgt = (jnp.bitwise_xor(me, LAYER_DIMS[L][p]),)
            dwin = pltpu.make_async_remote_copy(
                src_ref=wq.at[L, pl.ds(0, n)],
                dst_ref=wq.at[L, pl.ds(n, n)],
                send_sem=sw.at[L, p], recv_sem=rw.at[L, p],
                device_id=tgt, device_id_type=pl.DeviceIdType.MESH,
            )
            dwo = pltpu.make_async_remote_copy(
                src_ref=woq.at[L, pl.ds(0, n)],
                dst_ref=woq.at[L, pl.ds(n, n)],
                send_sem=swo.at[L, p], recv_sem=rwo.at[L, p],
                device_id=tgt, device_id_type=pl.DeviceIdType.MESH,
            )
            return dwin, dwo

        phases = [[None] * 3 for _ in range(NL)]
        for L in range(NL):
            phases[L][0] = make_phase(L, 0)
            phases[L][0][0].start()
            phases[L][0][1].start()
        for p in (1, 2):
            for L in range(NL):
                phases[L][p - 1][0].wait_recv()
                phases[L][p - 1][1].wait_recv()
                phases[L][p] = make_phase(L, p)
                phases[L][p][0].start()
                phases[L][p][1].start()

        def term(xcur, L, q):
            hid = jnp.maximum(
                jnp.dot(xcur, wq[L, q, :, :],
                        preferred_element_type=jnp.float32), 0.0)
            return jnp.dot(hid.astype(bf16), woq[L, q, :, :],
                           preferred_element_type=jnp.float32)

        acc = None
        for L in range(NL):
            xcur = xb[L, :, :]
            acc = term(xcur, L, 0)
            for q in range(1, 4):
                acc = acc + term(xcur, L, q)
            phases[L][2][0].wait_recv()
            phases[L][2][1].wait_recv()
            for q in range(4, N_DEV):
                acc = acc + term(xcur, L, q)
            if L < NL - 1:
                xb[L + 1, :, :] = acc.astype(bf16)

        ag_mine[...] = acc.astype(bf16)
        out_ref[pl.ds(me * b, b), :] = acc
        agsends = []
        for dd in range(1, N_DEV):
            r = pltpu.make_async_remote_copy(
                src_ref=ag_mine, dst_ref=agbuf.at[dd - 1],
                send_sem=sag.at[dd - 1], recv_sem=rag.at[dd - 1],
                device_id=(lax.rem(me + dd, N_DEV),),
                device_id_type=pl.DeviceIdType.MESH,
            )
            r.start()
            agsends.append(r)
        for dd in range(1, N_DEV):
            agsends[dd - 1].wait_recv()
            origin = lax.rem(me - dd + N_DEV, N_DEV)
            out_ref[pl.ds(origin * b, b), :] = agbuf[dd - 1, :, :].astype(
                jnp.float32)

        for L in range(NL):
            for p in range(3):
                phases[L][p][0].wait_send()
                phases[L][p][1].wait_send()
        for r in agsends:
            r.wait_send()

    return pl.pallas_call(
        body,
        out_shape=jax.ShapeDtypeStruct((N_DEV * b, d), jnp.float32),
        in_specs=[pl.BlockSpec(memory_space=pltpu.VMEM)] * 7,
        out_specs=pl.BlockSpec(memory_space=pltpu.VMEM),
        scratch_shapes=[
            pltpu.VMEM((NL, N_DEV, d, f), bf16),
            pltpu.VMEM((NL, N_DEV, f, d), bf16),
            pltpu.VMEM((N_DEV - 1, b, d), bf16),
            pltpu.VMEM((b, d), bf16),
            pltpu.VMEM((NL, b, d), bf16),
            pltpu.SemaphoreType.DMA((NL, 3)),
            pltpu.SemaphoreType.DMA((NL, 3)),
            pltpu.SemaphoreType.DMA((NL, 3)),
            pltpu.SemaphoreType.DMA((NL, 3)),
            pltpu.SemaphoreType.DMA((N_DEV - 1,)),
            pltpu.SemaphoreType.DMA((N_DEV - 1,)),
        ],
        compiler_params=pltpu.CompilerParams(collective_id=0),
    )(x, Win0, Wout0, Win1, Wout1, Win2, Wout2)


# device time: 81359 ns/iter; 4.1827x vs baseline; 1.0067x over previous
import jax
import jax.numpy as jnp
from jax import lax
from jax.experimental import pallas as pl
from jax.experimental.pallas import tpu as pltpu

N_DEV = 8
NL = 3
DIM_MASKS = (1, 3, 4)
LAYER_DIMS = tuple(
    tuple(DIM_MASKS[(L + p) % 3] for p in range(3)) for L in range(NL)
)


def kernel(x, Win0, Wout0, Win1, Wout1, Win2, Wout2):
    b, d = x.shape
    _, f = Win0.shape
    assert Wout0.shape == (f, d)
    bf16 = jnp.bfloat16
    hb = b // 2

    def body(x_ref, win0_ref, wout0_ref, win1_ref, wout1_ref, win2_ref,
             wout2_ref, out_ref,
             wpq, agbuf, ag_mine, xb,
             sw, rw, sag, rag):
        me = lax.axis_index("i")

        win_refs = (win0_ref, win1_ref, win2_ref)
        wout_refs = (wout0_ref, wout1_ref, wout2_ref)
        for L in range(NL):
            wpq[L, 0, 0, :, :] = win_refs[L][...].astype(bf16).T
            wpq[L, 0, 1, :, :] = wout_refs[L][...].astype(bf16)
        xb[0, :, :] = x_ref[...].astype(bf16)

        bar = pltpu.get_barrier_semaphore()
        for dd in range(1, N_DEV):
            pl.semaphore_signal(
                bar, inc=1, device_id=(lax.rem(me + dd, N_DEV),),
                device_id_type=pl.DeviceIdType.MESH,
            )
        pl.semaphore_wait(bar, N_DEV - 1)

        def make_phase(L, p):
            n = 1 << p
            return pltpu.make_async_remote_copy(
                src_ref=wpq.at[L, pl.ds(0, n)],
                dst_ref=wpq.at[L, pl.ds(n, n)],
                send_sem=sw.at[L, p], recv_sem=rw.at[L, p],
                device_id=(jnp.bitwise_xor(me, LAYER_DIMS[L][p]),),
                device_id_type=pl.DeviceIdType.MESH,
            )

        phases = [[None] * 3 for _ in range(NL)]
        for L in range(NL):
            phases[L][0] = make_phase(L, 0)
            phases[L][0].start()
        for p in (1, 2):
            for L in range(NL):
                phases[L][p - 1].wait_recv()
                phases[L][p] = make_phase(L, p)
                phases[L][p].start()

        def nt(a_rows, wt):
            return lax.dot_general(
                a_rows, wt, (((1,), (1,)), ((), ())),
                preferred_element_type=jnp.float32)

        def nn(h_rows, wo):
            return lax.dot_general(
                h_rows, wo, (((1,), (0,)), ((), ())),
                preferred_element_type=jnp.float32)

        def term(xcur, wt, wo):
            hid = jnp.maximum(nt(xcur, wt), 0.0)
            return nn(hid.astype(bf16), wo)

        x0 = xb[0, :, :]
        acc = term(x0, wpq[0, 0, 0], wpq[0, 0, 1])
        for q in range(1, 4):
            acc = acc + term(x0, wpq[0, q, 0], wpq[0, q, 1])
        phases[0][2].wait_recv()
        wt_hi = wpq[0, pl.ds(4, 4), 0].reshape(4 * f, d)
        wo_hi = wpq[0, pl.ds(4, 4), 1].reshape(4 * f, d)
        acc = acc + term(x0, wt_hi, wo_hi)
        xb[1, :, :] = acc.astype(bf16)

        phases[1][2].wait_recv()
        x1 = xb[1, :, :]
        wt_full = wpq[1, :, 0].reshape(N_DEV * f, d)
        wo_full = wpq[1, :, 1].reshape(N_DEV * f, d)
        acc = term(x1, wt_full, wo_full)
        xb[2, :, :] = acc.astype(bf16)

        phases[2][2].wait_recv()
        wt_full = wpq[2, :, 0].reshape(N_DEV * f, d)
        wo_full = wpq[2, :, 1].reshape(N_DEV * f, d)
        agsends = []
        for half in range(2):
            xh = xb[2, pl.ds(half * hb, hb), :]
            th = term(xh, wt_full, wo_full)
            out_ref[pl.ds(me * b + half * hb, hb), :] = th
            ag_mine[half, :, :] = th.astype(bf16)
            for dd in range(1, N_DEV):
                r = pltpu.make_async_remote_copy(
                    src_ref=ag_mine.at[half],
                    dst_ref=agbuf.at[dd - 1, half],
                    send_sem=sag.at[dd - 1, half],
                    recv_sem=rag.at[dd - 1, half],
                    device_id=(lax.rem(me + dd, N_DEV),),
                    device_id_type=pl.DeviceIdType.MESH,
                )
                r.start()
                agsends.append(r)
        for i, r in enumerate(agsends):
            r.wait_recv()
            dd, half = i % (N_DEV - 1) + 1, i // (N_DEV - 1)
            origin = lax.rem(me - dd + N_DEV, N_DEV)
            out_ref[pl.ds(origin * b + half * hb, hb), :] = agbuf[
                dd - 1, half, :, :].astype(jnp.float32)

        for L in range(NL):
            for p in range(3):
                phases[L][p].wait_send()
        for r in agsends:
            r.wait_send()

    return pl.pallas_call(
        body,
        out_shape=jax.ShapeDtypeStruct((N_DEV * b, d), jnp.float32),
        in_specs=[pl.BlockSpec(memory_space=pltpu.VMEM)] * 7,
        out_specs=pl.BlockSpec(memory_space=pltpu.VMEM),
        scratch_shapes=[
            pltpu.VMEM((NL, N_DEV, 2, f, d), bf16),
            pltpu.VMEM((N_DEV - 1, 2, hb, d), bf16),
            pltpu.VMEM((2, hb, d), bf16),
            pltpu.VMEM((NL, b, d), bf16),
            pltpu.SemaphoreType.DMA((NL, 3)),
            pltpu.SemaphoreType.DMA((NL, 3)),
            pltpu.SemaphoreType.DMA((N_DEV - 1, 2)),
            pltpu.SemaphoreType.DMA((N_DEV - 1, 2)),
        ],
        compiler_params=pltpu.CompilerParams(collective_id=0),
    )(x, Win0, Wout0, Win1, Wout1, Win2, Wout2)


# device time: 79582 ns/iter; 4.2761x vs baseline; 1.0223x over previous
import jax
import jax.numpy as jnp
from jax import lax
from jax.experimental import pallas as pl
from jax.experimental.pallas import tpu as pltpu

N_DEV = 8
NL = 3
DIM_MASKS = (1, 3, 4)
LAYER_DIMS = tuple(
    tuple(DIM_MASKS[(L + p) % 3] for p in range(3)) for L in range(NL)
)


def kernel(x, Win0, Wout0, Win1, Wout1, Win2, Wout2):
    b, d = x.shape
    _, f = Win0.shape
    assert Wout0.shape == (f, d)
    bf16 = jnp.bfloat16
    NQ = 4
    qb = b // NQ

    def body(x_ref, win0_ref, wout0_ref, win1_ref, wout1_ref, win2_ref,
             wout2_ref, out_ref,
             wpq, agbuf, ag_mine, xb,
             sw, rw, sag, rag):
        me = lax.axis_index("i")

        bar = pltpu.get_barrier_semaphore()
        for m in DIM_MASKS:
            pl.semaphore_signal(
                bar, inc=1, device_id=(jnp.bitwise_xor(me, m),),
                device_id_type=pl.DeviceIdType.MESH,
            )

        win_refs = (win0_ref, win1_ref, win2_ref)
        wout_refs = (wout0_ref, wout1_ref, wout2_ref)
        for L in range(NL):
            wpq[L, 0, 0, :, :] = win_refs[L][...].astype(bf16).T
            wpq[L, 0, 1, :, :] = wout_refs[L][...].astype(bf16)
        xb[0, :, :] = x_ref[...].astype(bf16)

        pl.semaphore_wait(bar, 3)

        def make_phase(L, p):
            n = 1 << p
            return pltpu.make_async_remote_copy(
                src_ref=wpq.at[L, pl.ds(0, n)],
                dst_ref=wpq.at[L, pl.ds(n, n)],
                send_sem=sw.at[L, p], recv_sem=rw.at[L, p],
                device_id=(jnp.bitwise_xor(me, LAYER_DIMS[L][p]),),
                device_id_type=pl.DeviceIdType.MESH,
            )

        phases = [[None] * 3 for _ in range(NL)]
        for L in range(NL):
            phases[L][0] = make_phase(L, 0)
            phases[L][0].start()
        for p in (1, 2):
            for L in range(NL):
                phases[L][p - 1].wait_recv()
                phases[L][p] = make_phase(L, p)
                phases[L][p].start()

        def nt(a_rows, wt):
            return lax.dot_general(
                a_rows, wt, (((1,), (1,)), ((), ())),
                preferred_element_type=jnp.float32)

        def nn(h_rows, wo):
            return lax.dot_general(
                h_rows, wo, (((1,), (0,)), ((), ())),
                preferred_element_type=jnp.float32)

        def term(xcur, wt, wo):
            hid = jnp.maximum(nt(xcur, wt), 0.0)
            return nn(hid.astype(bf16), wo)

        x0 = xb[0, :, :]
        acc = term(x0, wpq[0, 0, 0], wpq[0, 0, 1])
        for q in range(1, 4):
            acc = acc + term(x0, wpq[0, q, 0], wpq[0, q, 1])
        phases[0][2].wait_recv()
        wt_hi = wpq[0, pl.ds(4, 4), 0].reshape(4 * f, d)
        wo_hi = wpq[0, pl.ds(4, 4), 1].reshape(4 * f, d)
        acc = acc + term(x0, wt_hi, wo_hi)
        xb[1, :, :] = acc.astype(bf16)

        phases[1][2].wait_recv()
        x1 = xb[1, :, :]
        wt_full = wpq[1, :, 0].reshape(N_DEV * f, d)
        wo_full = wpq[1, :, 1].reshape(N_DEV * f, d)
        acc = term(x1, wt_full, wo_full)
        xb[2, :, :] = acc.astype(bf16)

        phases[2][2].wait_recv()
        wt_full = wpq[2, :, 0].reshape(N_DEV * f, d)
        wo_full = wpq[2, :, 1].reshape(N_DEV * f, d)
        agsends = []
        for qq in range(NQ):
            xh = xb[2, pl.ds(qq * qb, qb), :]
            th = term(xh, wt_full, wo_full)
            out_ref[pl.ds(me * b + qq * qb, qb), :] = th
            ag_mine[qq, :, :] = th.astype(bf16)
            for dd in range(1, N_DEV):
                r = pltpu.make_async_remote_copy(
                    src_ref=ag_mine.at[qq],
                    dst_ref=agbuf.at[dd - 1, qq],
                    send_sem=sag.at[dd - 1, qq],
                    recv_sem=rag.at[dd - 1, qq],
                    device_id=(lax.rem(me + dd, N_DEV),),
                    device_id_type=pl.DeviceIdType.MESH,
                )
                r.start()
                agsends.append(r)
        for i, r in enumerate(agsends):
            r.wait_recv()
            dd, qq = i % (N_DEV - 1) + 1, i // (N_DEV - 1)
            origin = lax.rem(me - dd + N_DEV, N_DEV)
            out_ref[pl.ds(origin * b + qq * qb, qb), :] = agbuf[
                dd - 1, qq, :, :].astype(jnp.float32)

        for L in range(NL):
            for p in range(3):
                phases[L][p].wait_send()
        for r in agsends:
            r.wait_send()

    return pl.pallas_call(
        body,
        out_shape=jax.ShapeDtypeStruct((N_DEV * b, d), jnp.float32),
        in_specs=[pl.BlockSpec(memory_space=pltpu.VMEM)] * 7,
        out_specs=pl.BlockSpec(memory_space=pltpu.VMEM),
        scratch_shapes=[
            pltpu.VMEM((NL, N_DEV, 2, f, d), bf16),
            pltpu.VMEM((N_DEV - 1, NQ, qb, d), bf16),
            pltpu.VMEM((NQ, qb, d), bf16),
            pltpu.VMEM((NL, b, d), bf16),
            pltpu.SemaphoreType.DMA((NL, 3)),
            pltpu.SemaphoreType.DMA((NL, 3)),
            pltpu.SemaphoreType.DMA((N_DEV - 1, NQ)),
            pltpu.SemaphoreType.DMA((N_DEV - 1, NQ)),
        ],
        compiler_params=pltpu.CompilerParams(collective_id=0),
    )(x, Win0, Wout0, Win1, Wout1, Win2, Wout2)
